# Initial kernel scaffold; baseline (speedup 1.0000x reference)
import jax
import jax.numpy as jnp
from jax.experimental import pallas as pl


def kernel(x, propagation_adj, filter_vals, W1, b1, W2, b2, filter_rows, filter_cols):
    raise NotImplementedError("write your pallas kernel here")


# speedup vs baseline: 3.9994x; 3.9994x over previous
import functools

import jax
import jax.numpy as jnp
from jax import lax
from jax.experimental import pallas as pl
from jax.experimental.pallas import tpu as pltpu
from jax.experimental.pallas import tpu_sc as plsc

_N = 10000
_E = 320000
_NC = 2
_NS = 16
_NW = _NC * _NS
_RPT = 320
_NPAD = _NW * _RPT
_RPC = _NS * _RPT
_B = 128


def _lower_bound(rows_hbm, bs_v, target):
    def step(_, carry):
        lo, hi = carry
        mid = jnp.minimum((lo + hi) // 2, _E - 1)
        base = (mid // 16) * 16
        pltpu.sync_copy(rows_hbm.at[pl.ds(base, 16)], bs_v.at[pl.ds(0, 16)])
        rv = bs_v[pl.ds(mid - base, 16)][0]
        valid = lo < hi
        less = rv < target
        lo = jnp.where(valid & less, mid + 1, lo)
        hi = jnp.where(valid & jnp.logical_not(less), mid, hi)
        return lo, hi

    lo, _ = lax.fori_loop(0, 19, step, (jnp.int32(0), jnp.int32(_E)))
    return lo


def _make_spmm(d, with_bias):
    mesh = plsc.VectorSubcoreMesh(
        core_axis_name="c", subcore_axis_name="s", num_cores=_NC, num_subcores=_NS
    )
    grp = d // 16

    def body(*refs):
        if with_bias:
            (h_hbm, rows_hbm, cols_hbm, vals_hbm, bias_hbm, out_hbm,
             bs_v, cols_v, lr_v, vals_v, g_v, init_v, bias_v, acc_sh, sem) = refs
        else:
            (h_hbm, rows_hbm, cols_hbm, vals_hbm, out_hbm,
             bs_v, cols_v, lr_v, vals_v, g_v, init_v, acc_sh, sem) = refs

        c = lax.axis_index("c")
        s = lax.axis_index("s")
        wid = c * _NS + s
        row_lo = wid * _RPT

        if with_bias:
            pltpu.sync_copy(bias_hbm, bias_v)
            ivecs = [bias_v[pl.ds(j * 16, 16)] for j in range(grp)]
        else:
            ivecs = [jnp.zeros((16,), jnp.float32)] * grp

        def init_row(r, carry):
            for j in range(grp):
                init_v[r, pl.ds(j * 16, 16)] = ivecs[j]
            return carry

        lax.fori_loop(0, _RPT, init_row, 0)
        pltpu.sync_copy(init_v, acc_sh.at[pl.ds(s * _RPT, _RPT)])

        e_start = _lower_bound(rows_hbm, bs_v, row_lo)
        e_end = _lower_bound(rows_hbm, bs_v, row_lo + _RPT)
        es_al = (e_start // 8) * 8
        nb = (e_end - es_al + _B - 1) // _B

        def block(b, carry):
            e0_nom = es_al + b * _B
            e0 = jnp.minimum(e0_nom, _E - _B)
            pltpu.sync_copy(cols_hbm.at[pl.ds(e0, _B)], cols_v)
            pltpu.sync_copy(vals_hbm.at[pl.ds(e0, _B)], vals_v.at[pl.ds(0, _B)])
            pltpu.sync_copy(rows_hbm.at[pl.ds(e0, _B)], lr_v)

            for g in range(_B // 16):
                lane_e = e0 + g * 16 + lax.iota(jnp.int32, 16)
                valid = (
                    (lane_e >= e_start) & (lane_e < e_end) & (lane_e >= e0_nom)
                )
                sl = pl.ds(g * 16, 16)
                cols_v[sl] = jnp.where(valid, cols_v[sl], 0)
                vals_v[sl] = jnp.where(valid, vals_v[sl], 0.0)
                lr = jnp.clip(lr_v[sl] - row_lo, 0, _RPT - 1) + s * _RPT
                lr_v[sl] = lr

            pltpu.async_copy(h_hbm.at[cols_v], g_v, sem).wait()

            def scale(e, carry2):
                v = vals_v[pl.ds(e, 16)][0]
                for j in range(grp):
                    sl = pl.ds(j * 16, 16)
                    g_v[e, sl] = g_v[e, sl] * v
                return carry2

            lax.fori_loop(0, _B, scale, 0)

            pltpu.sync_copy(g_v, acc_sh.at[lr_v], add=True)
            return carry

        lax.fori_loop(0, nb, block, 0)

        pltpu.sync_copy(
            acc_sh.at[pl.ds(s * _RPT, _RPT)], out_hbm.at[pl.ds(row_lo, _RPT)]
        )

    scratch = [
        pltpu.VMEM((32,), jnp.int32),
        pltpu.VMEM((_B,), jnp.int32),
        pltpu.VMEM((_B,), jnp.int32),
        pltpu.VMEM((_B + 16,), jnp.float32),
        pltpu.VMEM((_B, d), jnp.float32),
        pltpu.VMEM((_RPT, d), jnp.float32),
    ]
    if with_bias:
        scratch.append(pltpu.VMEM((d,), jnp.float32))
    scratch += [
        pltpu.VMEM_SHARED((_RPC, d), jnp.float32),
        pltpu.SemaphoreType.DMA,
    ]

    return pl.kernel(
        body,
        out_type=jax.ShapeDtypeStruct((_NPAD, d), jnp.float32),
        mesh=mesh,
        scratch_types=scratch,
        compiler_params=pltpu.CompilerParams(use_tc_tiling_on_sc=(d % 128 == 0)),
    )


def _tc_dense(h_ref, w1t_ref, b1_ref, w2t_ref, o_ref):
    h = h_ref[...]
    z = jnp.dot(h, w1t_ref[...], preferred_element_type=jnp.float32)
    z = jnp.maximum(z + b1_ref[...], 0.0)
    o_ref[...] = jnp.dot(z, w2t_ref[...], preferred_element_type=jnp.float32)


@jax.jit
def kernel(x, propagation_adj, filter_vals, W1, b1, W2, b2, filter_rows, filter_cols):
    del propagation_adj
    d_hid = W1.shape[0]
    n_cls = W2.shape[0]

    spmm1 = _make_spmm(d_hid, with_bias=False)
    spmm2 = _make_spmm(n_cls, with_bias=True)

    h1 = spmm1(x, filter_rows, filter_cols, filter_vals)

    t = pl.pallas_call(
        _tc_dense,
        out_shape=jax.ShapeDtypeStruct((_NPAD, n_cls), jnp.float32),
    )(h1, W1.T, b1[None, :], W2.T)

    outp = spmm2(t, filter_rows, filter_cols, filter_vals, b2)
    return outp[:_N]


# speedup vs baseline: 6.3723x; 1.5933x over previous
import jax
import jax.numpy as jnp
from jax import lax
from jax.experimental import pallas as pl
from jax.experimental.pallas import tpu as pltpu
from jax.experimental.pallas import tpu_sc as plsc

_N = 10000
_E = 320000
_NC = 2
_NS = 16
_NW = _NC * _NS
_RPT = 320
_NPAD = _NW * _RPT
_RPC = _NS * _RPT
_B = 256


def _lower_bound(rows_hbm, bs_v, target):
    def step(_, carry):
        lo, hi = carry
        mid = jnp.minimum((lo + hi) // 2, _E - 1)
        base = (mid // 16) * 16
        pltpu.sync_copy(rows_hbm.at[pl.ds(base, 16)], bs_v.at[pl.ds(0, 16)])
        rv = bs_v[pl.ds(mid - base, 16)][0]
        valid = lo < hi
        less = rv < target
        lo = jnp.where(valid & less, mid + 1, lo)
        hi = jnp.where(valid & jnp.logical_not(less), mid, hi)
        return lo, hi

    lo, _ = lax.fori_loop(0, 19, step, (jnp.int32(0), jnp.int32(_E)))
    return lo


def _make_spmm(d, with_bias):
    mesh = plsc.VectorSubcoreMesh(
        core_axis_name="c", subcore_axis_name="s", num_cores=_NC, num_subcores=_NS
    )
    grp = d // 16

    def body(*refs):
        if with_bias:
            (h_hbm, rows_hbm, cols_hbm, vals_hbm, bias_hbm, out_hbm,
             bs_v, bias_v,
             cols0, cols1, rows0, rows1, lr0, lr1, vals0, vals1, g0, g1,
             acc_sh, si0, si1, sg0, sg1, ss0, ss1) = refs
        else:
            (h_hbm, rows_hbm, cols_hbm, vals_hbm, out_hbm,
             bs_v,
             cols0, cols1, rows0, rows1, lr0, lr1, vals0, vals1, g0, g1,
             acc_sh, si0, si1, sg0, sg1, ss0, ss1) = refs

        cols = (cols0, cols1)
        rows_s = (rows0, rows1)
        lr = (lr0, lr1)
        vals = (vals0, vals1)
        g = (g0, g1)
        sem_i = (si0, si1)
        sem_g = (sg0, sg1)
        sem_s = (ss0, ss1)

        c = lax.axis_index("c")
        s = lax.axis_index("s")
        wid = c * _NS + s
        row_lo = wid * _RPT

        if with_bias:
            pltpu.sync_copy(bias_hbm, bias_v)
            ivecs = [bias_v[pl.ds(j * 16, 16)] for j in range(grp)]
        else:
            ivecs = [jnp.zeros((16,), jnp.float32)] * grp

        def init_row(r, carry):
            for j in range(grp):
                g0[r, pl.ds(j * 16, 16)] = ivecs[j]
            return carry

        lax.fori_loop(0, 64, init_row, 0)
        for k in range(_RPT // 64):
            pltpu.sync_copy(
                g0.at[pl.ds(0, 64)], acc_sh.at[pl.ds(s * _RPT + k * 64, 64)]
            )

        e_start = _lower_bound(rows_hbm, bs_v, row_lo)
        e_end = _lower_bound(rows_hbm, bs_v, row_lo + _RPT)
        es_al = (e_start // 8) * 8
        nb = (e_end - es_al + _B - 1) // _B

        def e0_of(b):
            return jnp.minimum(es_al + b * _B, _E - _B)

        def idx_start(b, j):
            e0 = e0_of(b)
            for h in range(2):
                pltpu.async_copy(
                    cols_hbm.at[pl.ds(e0 + h * 128, 128)], cols[j].at[h], sem_i[j]
                )
                pltpu.async_copy(
                    rows_hbm.at[pl.ds(e0 + h * 128, 128)], rows_s[j].at[h], sem_i[j]
                )
            pltpu.async_copy(
                vals_hbm.at[pl.ds(e0, _B)], vals[j].at[pl.ds(0, _B)], sem_i[j]
            )

        def idx_wait(j):
            for h in range(2):
                pltpu.make_async_copy(
                    cols_hbm.at[pl.ds(0, 128)], cols[j].at[h], sem_i[j]
                ).wait()
                pltpu.make_async_copy(
                    rows_hbm.at[pl.ds(0, 128)], rows_s[j].at[h], sem_i[j]
                ).wait()
            pltpu.make_async_copy(
                vals_hbm.at[pl.ds(0, _B)], vals[j].at[pl.ds(0, _B)], sem_i[j]
            ).wait()

        def mask(b, j):
            e0 = e0_of(b)
            e0_nom = es_al + b * _B
            for h in range(2):
                for gi in range(8):
                    lane_e = e0 + h * 128 + gi * 16 + lax.iota(jnp.int32, 16)
                    valid = (
                        (lane_e >= e_start) & (lane_e < e_end) & (lane_e >= e0_nom)
                    )
                    sl = pl.ds(gi * 16, 16)
                    cols[j][h, sl] = jnp.where(valid, cols[j][h, sl], 0)
                    fsl = pl.ds(h * 128 + gi * 16, 16)
                    vals[j][fsl] = jnp.where(valid, vals[j][fsl], 0.0)
                    lrv = jnp.clip(rows_s[j][h, sl] - row_lo, 0, _RPT - 1) + s * _RPT
                    lr[j][h, sl] = lrv

        def gather_start(j):
            for h in range(2):
                pltpu.async_copy(
                    h_hbm.at[cols[j].at[h]], g[j].at[pl.ds(h * 128, 128)], sem_g[j]
                )

        def gather_wait(j):
            for h in range(2):
                pltpu.make_async_copy(
                    h_hbm.at[pl.ds(0, 128)], g[j].at[pl.ds(h * 128, 128)], sem_g[j]
                ).wait()

        def scale(j):
            @plsc.parallel_loop(0, _B, unroll=4)
            def _(e):
                v = vals[j][pl.ds(e, 16)][0]
                for gi in range(grp):
                    sl = pl.ds(gi * 16, 16)
                    g[j][e, sl] = g[j][e, sl] * v

        def scatter_start(j):
            for h in range(2):
                pltpu.async_copy(
                    g[j].at[pl.ds(h * 128, 128)],
                    acc_sh.at[lr[j].at[h]],
                    sem_s[j],
                    add=True,
                )

        def scatter_wait(j):
            for h in range(2):
                pltpu.make_async_copy(
                    h_hbm.at[pl.ds(0, 128)], g[j].at[pl.ds(h * 128, 128)], sem_s[j]
                ).wait()

        @pl.when(nb > 0)
        def _():
            idx_start(0, 0)

        @pl.when(nb > 1)
        def _():
            idx_start(1, 1)

        @pl.when(nb > 0)
        def _():
            idx_wait(0)
            mask(0, 0)
            gather_start(0)

        def outer(i, carry):
            for jj in range(2):
                b = 2 * i + jj
                j, j2 = jj, 1 - jj

                @pl.when(b < nb)
                def _():
                    gather_wait(j)
                    scale(j)
                    scatter_start(j)

                    @pl.when(b + 1 < nb)
                    def _():
                        idx_wait(j2)

                        @pl.when(b > 0)
                        def _():
                            scatter_wait(j2)

                        mask(b + 1, j2)
                        gather_start(j2)

                    @pl.when(b + 2 < nb)
                    def _():
                        idx_start(b + 2, j)

            return carry

        lax.fori_loop(0, (nb + 1) // 2, outer, 0)

        @pl.when(nb >= 1)
        def _():
            scatter_wait(0)

        @pl.when(nb >= 2)
        def _():
            scatter_wait(1)

        pltpu.sync_copy(
            acc_sh.at[pl.ds(s * _RPT, _RPT)], out_hbm.at[pl.ds(row_lo, _RPT)]
        )

    scratch = [
        pltpu.VMEM((32,), jnp.int32),
    ]
    if with_bias:
        scratch.append(pltpu.VMEM((d,), jnp.float32))
    scratch += [
        pltpu.VMEM((2, 128), jnp.int32),
        pltpu.VMEM((2, 128), jnp.int32),
        pltpu.VMEM((2, 128), jnp.int32),
        pltpu.VMEM((2, 128), jnp.int32),
        pltpu.VMEM((2, 128), jnp.int32),
        pltpu.VMEM((2, 128), jnp.int32),
        pltpu.VMEM((_B + 16,), jnp.float32),
        pltpu.VMEM((_B + 16,), jnp.float32),
        pltpu.VMEM((_B, d), jnp.float32),
        pltpu.VMEM((_B, d), jnp.float32),
        pltpu.VMEM_SHARED((_RPC, d), jnp.float32),
        pltpu.SemaphoreType.DMA,
        pltpu.SemaphoreType.DMA,
        pltpu.SemaphoreType.DMA,
        pltpu.SemaphoreType.DMA,
        pltpu.SemaphoreType.DMA,
        pltpu.SemaphoreType.DMA,
    ]

    return pl.kernel(
        body,
        out_type=jax.ShapeDtypeStruct((_NPAD, d), jnp.float32),
        mesh=mesh,
        scratch_types=scratch,
        compiler_params=pltpu.CompilerParams(use_tc_tiling_on_sc=(d % 128 == 0)),
    )


def _tc_dense(h_ref, w1t_ref, b1_ref, w2t_ref, o_ref):
    h = h_ref[...]
    z = jnp.dot(h, w1t_ref[...], preferred_element_type=jnp.float32)
    z = jnp.maximum(z + b1_ref[...], 0.0)
    o_ref[...] = jnp.dot(z, w2t_ref[...], preferred_element_type=jnp.float32)


@jax.jit
def kernel(x, propagation_adj, filter_vals, W1, b1, W2, b2, filter_rows, filter_cols):
    del propagation_adj
    d_hid = W1.shape[0]
    n_cls = W2.shape[0]

    spmm1 = _make_spmm(d_hid, with_bias=False)
    spmm2 = _make_spmm(n_cls, with_bias=True)

    h1 = spmm1(x, filter_rows, filter_cols, filter_vals)

    t = pl.pallas_call(
        _tc_dense,
        out_shape=jax.ShapeDtypeStruct((_NPAD, n_cls), jnp.float32),
    )(h1, W1.T, b1[None, :], W2.T)

    outp = spmm2(t, filter_rows, filter_cols, filter_vals, b2)
    return outp[:_N]


# speedup vs baseline: 8.5811x; 1.3466x over previous
import jax
import jax.numpy as jnp
from jax import lax
from jax.experimental import pallas as pl
from jax.experimental.pallas import tpu as pltpu
from jax.experimental.pallas import tpu_sc as plsc

_N = 10000
_E = 320000
_NC = 2
_NS = 16
_NW = _NC * _NS
_RPT = 320
_NPAD = _NW * _RPT
_RPC = _NS * _RPT
_B = 128
_NSLOT = 4


def _lower_bound(rows_hbm, bs_v, target):
    def step(_, carry):
        lo, hi = carry
        mid = jnp.minimum((lo + hi) // 2, _E - 1)
        base = (mid // 16) * 16
        pltpu.sync_copy(rows_hbm.at[pl.ds(base, 16)], bs_v.at[pl.ds(0, 16)])
        rv = bs_v[pl.ds(mid - base, 16)][0]
        valid = lo < hi
        less = rv < target
        lo = jnp.where(valid & less, mid + 1, lo)
        hi = jnp.where(valid & jnp.logical_not(less), mid, hi)
        return lo, hi

    lo, _ = lax.fori_loop(0, 19, step, (jnp.int32(0), jnp.int32(_E)))
    return lo


def _make_spmm(d, with_bias, emit_offs):
    mesh = plsc.VectorSubcoreMesh(
        core_axis_name="c", subcore_axis_name="s", num_cores=_NC, num_subcores=_NS
    )
    grp = d // 16

    def body(*refs):
        it = iter(refs)
        h_hbm = next(it)
        rows_hbm = next(it)
        cols_hbm = next(it)
        vals_hbm = next(it)
        offs_in = None if emit_offs else next(it)
        bias_hbm = next(it) if with_bias else None
        out_hbm = next(it)
        offs_out = next(it) if emit_offs else None
        bs_v = next(it)
        bias_v = next(it) if with_bias else None
        cols = [next(it) for _ in range(_NSLOT)]
        rows_s = [next(it) for _ in range(_NSLOT)]
        lr = [next(it) for _ in range(_NSLOT)]
        vals = [next(it) for _ in range(_NSLOT)]
        g = [next(it) for _ in range(_NSLOT)]
        acc_sh = next(it)
        sem_i = [next(it) for _ in range(_NSLOT)]
        sem_g = [next(it) for _ in range(_NSLOT)]
        sem_s = [next(it) for _ in range(_NSLOT)]

        c = lax.axis_index("c")
        s = lax.axis_index("s")
        wid = c * _NS + s
        row_lo = wid * _RPT

        if with_bias:
            pltpu.sync_copy(bias_hbm, bias_v)
            ivecs = [bias_v[pl.ds(j * 16, 16)] for j in range(grp)]
        else:
            ivecs = [jnp.zeros((16,), jnp.float32)] * grp

        def init_row(r, carry):
            for j in range(grp):
                g[0][r, pl.ds(j * 16, 16)] = ivecs[j]
            return carry

        lax.fori_loop(0, 64, init_row, 0)
        for k in range(_RPT // 64):
            pltpu.sync_copy(
                g[0].at[pl.ds(0, 64)], acc_sh.at[pl.ds(s * _RPT + k * 64, 64)]
            )

        if emit_offs:
            e_start = _lower_bound(rows_hbm, bs_v, row_lo)
            e_end = _lower_bound(rows_hbm, bs_v, row_lo + _RPT)
            vec = jnp.where(lax.iota(jnp.int32, 16) == 0, e_start, e_end)
            bs_v[pl.ds(0, 16)] = vec
            pltpu.sync_copy(bs_v.at[pl.ds(0, 16)], offs_out.at[pl.ds(wid * 16, 16)])
        else:
            pltpu.sync_copy(offs_in.at[pl.ds(wid * 16, 16)], bs_v.at[pl.ds(0, 16)])
            e_start = bs_v[pl.ds(0, 16)][0]
            e_end = bs_v[pl.ds(1, 16)][0]

        es_al = (e_start // 8) * 8
        nb = (e_end - es_al + _B - 1) // _B

        def e0_of(b):
            return jnp.minimum(es_al + b * _B, _E - _B)

        def idx_start(b, j):
            e0 = e0_of(b)
            pltpu.async_copy(cols_hbm.at[pl.ds(e0, _B)], cols[j], sem_i[j])
            pltpu.async_copy(rows_hbm.at[pl.ds(e0, _B)], rows_s[j], sem_i[j])
            pltpu.async_copy(
                vals_hbm.at[pl.ds(e0, _B)], vals[j].at[pl.ds(0, _B)], sem_i[j]
            )

        def idx_wait(j):
            pltpu.make_async_copy(cols_hbm.at[pl.ds(0, _B)], cols[j], sem_i[j]).wait()
            pltpu.make_async_copy(rows_hbm.at[pl.ds(0, _B)], rows_s[j], sem_i[j]).wait()
            pltpu.make_async_copy(
                vals_hbm.at[pl.ds(0, _B)], vals[j].at[pl.ds(0, _B)], sem_i[j]
            ).wait()

        def mask(b, j):
            e0 = e0_of(b)
            e0_nom = es_al + b * _B
            for gi in range(_B // 16):
                lane_e = e0 + gi * 16 + lax.iota(jnp.int32, 16)
                valid = (lane_e >= e_start) & (lane_e < e_end) & (lane_e >= e0_nom)
                sl = pl.ds(gi * 16, 16)
                cols[j][sl] = jnp.where(valid, cols[j][sl], 0)
                vals[j][sl] = jnp.where(valid, vals[j][sl], 0.0)
                lrv = jnp.clip(rows_s[j][sl] - row_lo, 0, _RPT - 1) + s * _RPT
                lr[j][0, sl] = lrv

        def gather_start(j):
            pltpu.async_copy(h_hbm.at[cols[j]], g[j], sem_g[j])

        def gather_wait(j):
            pltpu.make_async_copy(h_hbm.at[pl.ds(0, _B)], g[j], sem_g[j]).wait()

        def scale(j):
            @plsc.parallel_loop(0, _B, unroll=4)
            def _(e):
                v = vals[j][pl.ds(e, 16)][0]
                for gi in range(grp):
                    sl = pl.ds(gi * 16, 16)
                    g[j][e, sl] = g[j][e, sl] * v

        def scatter_start(j):
            pltpu.async_copy(g[j], acc_sh.at[lr[j].at[0]], sem_s[j], add=True)

        def scatter_wait(j):
            pltpu.make_async_copy(h_hbm.at[pl.ds(0, _B)], g[j], sem_s[j]).wait()

        for p in range(_NSLOT - 1):
            @pl.when(nb > p)
            def _(p=p):
                idx_start(p, p)

        @pl.when(nb > 0)
        def _():
            idx_wait(0)
            mask(0, 0)
            gather_start(0)

        def outer(i, carry):
            for jj in range(_NSLOT):
                b = _NSLOT * i + jj
                j = jj
                j1 = (jj + 1) % _NSLOT

                @pl.when(b < nb)
                def _():
                    gather_wait(j)

                    @pl.when(b + 1 < nb)
                    def _():
                        idx_wait(j1)

                        @pl.when(b >= _NSLOT - 1)
                        def _():
                            scatter_wait(j1)

                        mask(b + 1, j1)
                        gather_start(j1)

                    scale(j)
                    scatter_start(j)

                    @pl.when(b + _NSLOT - 1 < nb)
                    def _():
                        idx_start(b + _NSLOT - 1, (jj + _NSLOT - 1) % _NSLOT)

            return carry

        lax.fori_loop(0, (nb + _NSLOT - 1) // _NSLOT, outer, 0)

        for jj in range(_NSLOT):
            @pl.when(nb > jj)
            def _(jj=jj):
                scatter_wait(jj)

        pltpu.sync_copy(
            acc_sh.at[pl.ds(s * _RPT, _RPT)], out_hbm.at[pl.ds(row_lo, _RPT)]
        )

    scratch = [
        pltpu.VMEM((32,), jnp.int32),
    ]
    if with_bias:
        scratch.append(pltpu.VMEM((d,), jnp.float32))
    scratch += [pltpu.VMEM((_B,), jnp.int32) for _ in range(_NSLOT)]
    scratch += [pltpu.VMEM((_B,), jnp.int32) for _ in range(_NSLOT)]
    scratch += [pltpu.VMEM((1, _B), jnp.int32) for _ in range(_NSLOT)]
    scratch += [pltpu.VMEM((_B + 16,), jnp.float32) for _ in range(_NSLOT)]
    scratch += [pltpu.VMEM((_B, d), jnp.float32) for _ in range(_NSLOT)]
    scratch += [pltpu.VMEM_SHARED((_RPC, d), jnp.float32)]
    scratch += [pltpu.SemaphoreType.DMA for _ in range(3 * _NSLOT)]

    out_type = jax.ShapeDtypeStruct((_NPAD, d), jnp.float32)
    if emit_offs:
        out_type = [out_type, jax.ShapeDtypeStruct((_NW * 16,), jnp.int32)]

    return pl.kernel(
        body,
        out_type=out_type,
        mesh=mesh,
        scratch_types=scratch,
        compiler_params=pltpu.CompilerParams(use_tc_tiling_on_sc=(d % 128 == 0)),
    )


def _tc_dense(h_ref, w1t_ref, b1_ref, w2t_ref, o_ref):
    h = h_ref[...]
    z = jnp.dot(h, w1t_ref[...], preferred_element_type=jnp.float32)
    z = jnp.maximum(z + b1_ref[...], 0.0)
    o_ref[...] = jnp.dot(z, w2t_ref[...], preferred_element_type=jnp.float32)


@jax.jit
def kernel(x, propagation_adj, filter_vals, W1, b1, W2, b2, filter_rows, filter_cols):
    del propagation_adj
    d_hid = W1.shape[0]
    n_cls = W2.shape[0]

    spmm1 = _make_spmm(d_hid, with_bias=False, emit_offs=True)
    spmm2 = _make_spmm(n_cls, with_bias=True, emit_offs=False)

    h1, offs = spmm1(x, filter_rows, filter_cols, filter_vals)

    t = pl.pallas_call(
        _tc_dense,
        out_shape=jax.ShapeDtypeStruct((_NPAD, n_cls), jnp.float32),
    )(h1, W1.T, b1[None, :], W2.T)

    outp = spmm2(t, filter_rows, filter_cols, filter_vals, offs, b2)
    return outp[:_N]
